# baseline (speedup 1.0000x reference)
import jax
import jax.numpy as jnp
from jax.experimental import pallas as pl
from jax.experimental.pallas import tpu as pltpu

N = 10000
IN_CH = 512
NUM_RES = 4
HID = 1024
BN = 1000
NSTRIPE = IN_CH // 128
ROWS = NUM_RES * NSTRIPE


def _head_kernel(x_ref, wl_ref, W1_ref, b1_ref, w2_ref, bias_ref, out_ref):
    xlast = x_ref[pl.ds(NUM_RES - 1, NSTRIPE * BN, NUM_RES), :]
    xlast = xlast.reshape(BN, IN_CH)
    h = jnp.dot(xlast.astype(jnp.bfloat16), W1_ref[...],
                preferred_element_type=jnp.float32)
    h = h + b1_ref[...]
    h = h * jax.nn.sigmoid(h)
    acc = jnp.sum(h * w2_ref[...], axis=1, keepdims=True)
    ph = None
    for r in range(NUM_RES - 1):
        xr = x_ref[pl.ds(r, NSTRIPE * BN, NUM_RES), :]
        pt = xr.reshape(BN, NSTRIPE, 128) * wl_ref[r][None, :, :]
        ph = pt if ph is None else ph + pt
    acc = acc + jnp.sum(ph, axis=(1, 2), keepdims=False)[:, None]
    out_ref[...] = acc + bias_ref[...]


def kernel(embedding_0, W_lin, b_lin, W1, b1, W2, b2):
    x = jnp.squeeze(embedding_0, 2)
    x = x.reshape(N, NSTRIPE, 128, NUM_RES)
    x = jnp.transpose(x, (0, 1, 3, 2))
    x = x.reshape(N * ROWS, 128)
    wl = jnp.concatenate(
        [W_lin[:, :, 0], jnp.zeros((1, IN_CH), jnp.float32)], axis=0)
    wl = wl.reshape(NUM_RES, NSTRIPE, 128)
    bias = (jnp.sum(b_lin) + b2[0]).reshape(1, 1)

    out = pl.pallas_call(
        _head_kernel,
        grid=(N // BN,),
        in_specs=[
            pl.BlockSpec((ROWS * BN, 128), lambda i: (i, 0)),
            pl.BlockSpec((NUM_RES, NSTRIPE, 128), lambda i: (0, 0, 0)),
            pl.BlockSpec((IN_CH, HID), lambda i: (0, 0)),
            pl.BlockSpec((1, HID), lambda i: (0, 0)),
            pl.BlockSpec((1, HID), lambda i: (0, 0)),
            pl.BlockSpec((1, 1), lambda i: (0, 0)),
        ],
        out_specs=pl.BlockSpec((BN, 1), lambda i: (i, 0)),
        out_shape=jax.ShapeDtypeStruct((N, 1), jnp.float32),
        compiler_params=pltpu.CompilerParams(dimension_semantics=("parallel",)),
    )(x, wl, W1.astype(jnp.bfloat16), b1.reshape(1, HID),
      W2.reshape(1, HID), bias)
    return out.reshape(N)
